# baseline (speedup 1.0000x reference)
import jax
import jax.numpy as jnp
from jax import lax
from jax.experimental import pallas as pl
from jax.experimental.pallas import tpu as pltpu
from jax.experimental.pallas import tpu_sc as plsc

_VOCAB = 12
_NC = 2
_NS = 16
_NW = _NC * _NS
_CHUNK = 2048


_TROW = 16


def _table_body(e_ref, p_ref, b_ref, t_ref):
    t = lax.dot_general(
        e_ref[...], p_ref[...], (((1,), (1,)), ((), ())),
        preferred_element_type=jnp.float32,
    )
    t_ref[...] = jnp.concatenate(
        [t + b_ref[...], jnp.zeros((_VOCAB, _TROW - _VOCAB), jnp.float32)],
        axis=1,
    )


def _fused_table(embed_weight, proj_weight, proj_bias):
    return pl.pallas_call(
        _table_body,
        out_shape=jax.ShapeDtypeStruct((_VOCAB, _TROW), jnp.float32),
    )(embed_weight, proj_weight, proj_bias.reshape(1, _VOCAB))


def _make_lookup(n_tokens):
    per_w = n_tokens // _NW
    nchunks = per_w // _CHUNK
    assert per_w * _NW == n_tokens and nchunks * _CHUNK == per_w

    mesh = plsc.VectorSubcoreMesh(
        core_axis_name="c", subcore_axis_name="s",
        num_cores=_NC, num_subcores=_NS,
    )

    def body(ids_hbm, tab_hbm, out_hbm, idx_v, rows_v, sem):
        wid = lax.axis_index("s") * _NC + lax.axis_index("c")

        def step(c, carry):
            base = wid * per_w + c * _CHUNK
            pltpu.sync_copy(ids_hbm.at[pl.ds(base, _CHUNK)], idx_v)
            pltpu.async_copy(tab_hbm.at[idx_v], rows_v, sem).wait()
            pltpu.sync_copy(rows_v, out_hbm.at[pl.ds(base, _CHUNK)])
            return carry

        lax.fori_loop(0, nchunks, step, 0)

    return pl.kernel(
        body,
        out_type=jax.ShapeDtypeStruct((n_tokens, _TROW), jnp.float32),
        mesh=mesh,
        scratch_types=[
            pltpu.VMEM((_CHUNK,), jnp.int32),
            pltpu.VMEM((_CHUNK, _TROW), jnp.float32),
            pltpu.SemaphoreType.DMA,
        ],
        compiler_params=pltpu.CompilerParams(use_tc_tiling_on_sc=False),
    )


@jax.jit
def kernel(input_ids, embed_weight, proj_weight, proj_bias):
    b, s = input_ids.shape
    n = b * s
    tab = _fused_table(embed_weight, proj_weight, proj_bias)
    out = _make_lookup(n)(input_ids.reshape(n), tab)
    return out[:, :_VOCAB].reshape(b, s, _VOCAB)


# speedup vs baseline: 4.2314x; 4.2314x over previous
import jax
import jax.numpy as jnp
from jax import lax
from jax.experimental import pallas as pl
from jax.experimental.pallas import tpu as pltpu
from jax.experimental.pallas import tpu_sc as plsc

_VOCAB = 12
_NC = 2
_NS = 16
_NW = _NC * _NS
_CHUNK = 2048
_QCHUNK = _CHUNK // 4
_NQ = _VOCAB ** 4
_QROW = 4 * _VOCAB


def _qtab_body(e_ref, p_ref, b_ref, q_ref):
    t = lax.dot_general(
        e_ref[...], p_ref[...], (((1,), (1,)), ((), ())),
        preferred_element_type=jnp.float32,
    ) + b_ref[...]
    v = _VOCAB
    r_a = jnp.broadcast_to(t[:, None, :], (v, v * v * v, v)).reshape(_NQ, v)
    x = jnp.broadcast_to(t[:, None, :], (v, v * v, v)).reshape(v * v * v, v)
    r_b = jnp.broadcast_to(x[None], (v, v * v * v, v)).reshape(_NQ, v)
    y = jnp.broadcast_to(t[:, None, :], (v, v, v)).reshape(v * v, v)
    r_c = jnp.broadcast_to(y[None], (v * v, v * v, v)).reshape(_NQ, v)
    r_d = jnp.broadcast_to(t[None], (v * v * v, v, v)).reshape(_NQ, v)
    q_ref[...] = jnp.concatenate([r_a, r_b, r_c, r_d], axis=1)


def _quad_table(embed_weight, proj_weight, proj_bias):
    return pl.pallas_call(
        _qtab_body,
        out_shape=jax.ShapeDtypeStruct((_NQ, _QROW), jnp.float32),
    )(embed_weight, proj_weight, proj_bias.reshape(1, _VOCAB))


def _make_lookup(n_tokens):
    per_w = n_tokens // _NW
    per_wq = per_w // 4
    nchunks = per_w // _CHUNK
    assert per_w * _NW == n_tokens and nchunks * _CHUNK == per_w

    mesh = plsc.VectorSubcoreMesh(
        core_axis_name="c", subcore_axis_name="s",
        num_cores=_NC, num_subcores=_NS,
    )

    def body(ids_hbm, qtab_hbm, out_hbm, idx_v, qidx_v, rows_v, sem):
        wid = lax.axis_index("s") * _NC + lax.axis_index("c")
        iota4 = lax.iota(jnp.int32, 16) * 4

        def step(c, carry):
            tbase = wid * per_w + c * _CHUNK
            qbase = wid * per_wq + c * _QCHUNK
            pltpu.sync_copy(ids_hbm.at[pl.ds(tbase, _CHUNK)], idx_v)

            def qstep(j, carry2):
                t0 = iota4 + j * 64
                a = plsc.load_gather(idx_v, [t0])
                b = plsc.load_gather(idx_v, [t0 + 1])
                cc = plsc.load_gather(idx_v, [t0 + 2])
                d = plsc.load_gather(idx_v, [t0 + 3])
                q = ((a * _VOCAB + b) * _VOCAB + cc) * _VOCAB + d
                qidx_v[pl.ds(j * 16, 16)] = q
                return carry2

            lax.fori_loop(0, _QCHUNK // 16, qstep, 0)
            pltpu.async_copy(qtab_hbm.at[qidx_v], rows_v, sem).wait()
            pltpu.sync_copy(rows_v, out_hbm.at[pl.ds(qbase, _QCHUNK)])
            return carry

        lax.fori_loop(0, nchunks, step, 0)

    return pl.kernel(
        body,
        out_type=jax.ShapeDtypeStruct((n_tokens // 4, _QROW), jnp.float32),
        mesh=mesh,
        scratch_types=[
            pltpu.VMEM((_CHUNK,), jnp.int32),
            pltpu.VMEM((_QCHUNK,), jnp.int32),
            pltpu.VMEM((_QCHUNK, _QROW), jnp.float32),
            pltpu.SemaphoreType.DMA,
        ],
        compiler_params=pltpu.CompilerParams(
            use_tc_tiling_on_sc=False, needs_layout_passes=False,
        ),
    )


@jax.jit
def kernel(input_ids, embed_weight, proj_weight, proj_bias):
    b, s = input_ids.shape
    n = b * s
    qtab = _quad_table(embed_weight, proj_weight, proj_bias)
    out = _make_lookup(n)(input_ids.reshape(n), qtab)
    return out.reshape(b, s, _VOCAB)


# speedup vs baseline: 4.2873x; 1.0132x over previous
import jax
import jax.numpy as jnp
from jax import lax
from jax.experimental import pallas as pl
from jax.experimental.pallas import tpu as pltpu
from jax.experimental.pallas import tpu_sc as plsc

_VOCAB = 12
_NC = 2
_NS = 16
_NW = _NC * _NS
_CHUNK = 2048
_QCHUNK = _CHUNK // 4
_NQ = _VOCAB ** 4
_QROW = 4 * _VOCAB


def _qtab_body(e_ref, p_ref, b_ref, q_ref):
    t = lax.dot_general(
        e_ref[...], p_ref[...], (((1,), (1,)), ((), ())),
        preferred_element_type=jnp.float32,
    ) + b_ref[...]
    v = _VOCAB
    r_a = jnp.broadcast_to(t[:, None, :], (v, v * v * v, v)).reshape(_NQ, v)
    x = jnp.broadcast_to(t[:, None, :], (v, v * v, v)).reshape(v * v * v, v)
    r_b = jnp.broadcast_to(x[None], (v, v * v * v, v)).reshape(_NQ, v)
    y = jnp.broadcast_to(t[:, None, :], (v, v, v)).reshape(v * v, v)
    r_c = jnp.broadcast_to(y[None], (v * v, v * v, v)).reshape(_NQ, v)
    r_d = jnp.broadcast_to(t[None], (v * v * v, v, v)).reshape(_NQ, v)
    q_ref[...] = jnp.concatenate([r_a, r_b, r_c, r_d], axis=1)


def _quad_table(embed_weight, proj_weight, proj_bias):
    return pl.pallas_call(
        _qtab_body,
        out_shape=jax.ShapeDtypeStruct((_NQ, _QROW), jnp.float32),
    )(embed_weight, proj_weight, proj_bias.reshape(1, _VOCAB))


def _make_lookup(n_tokens):
    per_w = n_tokens // _NW
    per_wq = per_w // 4
    nchunks = per_w // _CHUNK
    assert per_w * _NW == n_tokens and nchunks * _CHUNK == per_w

    mesh = plsc.VectorSubcoreMesh(
        core_axis_name="c", subcore_axis_name="s",
        num_cores=_NC, num_subcores=_NS,
    )

    def body(ids_hbm, qtab_hbm, out_hbm, idx_v, qidx_v, rows_v, qtab_sh, sem):
        wid = lax.axis_index("s") * _NC + lax.axis_index("c")
        iota4 = lax.iota(jnp.int32, 16) * 4

        @pl.when(lax.axis_index("s") == 0)
        def _stage():
            pltpu.sync_copy(qtab_hbm, qtab_sh)

        plsc.subcore_barrier()

        def step(c, carry):
            tbase = wid * per_w + c * _CHUNK
            qbase = wid * per_wq + c * _QCHUNK
            pltpu.sync_copy(ids_hbm.at[pl.ds(tbase, _CHUNK)], idx_v)

            def qstep(j, carry2):
                t0 = iota4 + j * 64
                a = plsc.load_gather(idx_v, [t0])
                b = plsc.load_gather(idx_v, [t0 + 1])
                cc = plsc.load_gather(idx_v, [t0 + 2])
                d = plsc.load_gather(idx_v, [t0 + 3])
                q = ((a * _VOCAB + b) * _VOCAB + cc) * _VOCAB + d
                qidx_v[pl.ds(j * 16, 16)] = q
                return carry2

            lax.fori_loop(0, _QCHUNK // 16, qstep, 0)
            pltpu.async_copy(qtab_sh.at[qidx_v], rows_v, sem).wait()
            pltpu.sync_copy(rows_v, out_hbm.at[pl.ds(qbase, _QCHUNK)])
            return carry

        lax.fori_loop(0, nchunks, step, 0)

    return pl.kernel(
        body,
        out_type=jax.ShapeDtypeStruct((n_tokens // 4, _QROW), jnp.float32),
        mesh=mesh,
        scratch_types=[
            pltpu.VMEM((_CHUNK,), jnp.int32),
            pltpu.VMEM((_QCHUNK,), jnp.int32),
            pltpu.VMEM((_QCHUNK, _QROW), jnp.float32),
            pltpu.VMEM_SHARED((_NQ, _QROW), jnp.float32),
            pltpu.SemaphoreType.DMA,
        ],
        compiler_params=pltpu.CompilerParams(
            use_tc_tiling_on_sc=False, needs_layout_passes=False,
        ),
    )


@jax.jit
def kernel(input_ids, embed_weight, proj_weight, proj_bias):
    b, s = input_ids.shape
    n = b * s
    qtab = _quad_table(embed_weight, proj_weight, proj_bias)
    out = _make_lookup(n)(input_ids.reshape(n), qtab)
    return out.reshape(b, s, _VOCAB)
